# baseline (speedup 1.0000x reference)
import functools

import jax
import jax.numpy as jnp
from jax import lax
from jax.experimental import pallas as pl
from jax.experimental.pallas import tpu as pltpu
from jax.experimental.pallas import tpu_sc as plsc

_NS = 16
_N = 10000
_E = 160000
_D = 128
_SH = 9
_DE = 64

_CH = 128
_NCH = _E // _CH
_NW = 32
_NSL = 40
_RPS = 624
_RTAIL = _N - 16 * _RPS

_mesh = plsc.VectorSubcoreMesh(core_axis_name="c", subcore_axis_name="s")


def _slot_valid(c0, t):
    return ((c0 + t) < _NCH) & (t < _NSL)


def _preload_idx(src2, dst2, idx_s, idx_d, c0):
    pltpu.sync_copy(src2.at[pl.ds(c0, _NSL)], idx_s)
    pltpu.sync_copy(dst2.at[pl.ds(c0, _NSL)], idx_d)


def _make_sc_gather(stripe):
  @functools.partial(
      pl.kernel,
      mesh=_mesh,
      out_type=jax.ShapeDtypeStruct((_E, 48), jnp.float32),
      scratch_types=[
          pltpu.VMEM((_NSL, _CH), jnp.int32),
          pltpu.VMEM((_NSL, _CH), jnp.int32),
          pltpu.VMEM((_CH, _D), jnp.float32),
          pltpu.VMEM((_CH, _D), jnp.float32),
          pltpu.VMEM((_CH, _D), jnp.float32),
          pltpu.VMEM((_CH, _D), jnp.float32),
          pltpu.VMEM((_CH, 48), jnp.float32),
          pltpu.VMEM((_CH, 48), jnp.float32),
          pltpu.SemaphoreType.DMA,
          pltpu.SemaphoreType.DMA,
          pltpu.SemaphoreType.DMA,
          pltpu.SemaphoreType.DMA,
      ],
  )
  def _sc_gather(tbl, src2, dst2, s_out, idx_s, idx_d,
                 buf_s0, buf_s1, buf_d0, buf_d1, buf_u0, buf_u1,
                 sem_s0, sem_s1, sem_d0, sem_d1):
    wid = lax.axis_index("s") * 2 + lax.axis_index("c")
    c0 = wid * _NSL
    _preload_idx(src2, dst2, idx_s, idx_d, c0)

    def slot(j):
        return (j // 5) * 10 + stripe * 5 + j % 5

    def start(t, buf_s, buf_d, sem_s, sem_d):
        pltpu.async_copy(tbl.at[idx_s.at[t]], buf_s, sem_s)
        pltpu.async_copy(tbl.at[idx_d.at[t]], buf_d, sem_d)

    def wait(buf_s, buf_d, sem_s, sem_d):
        pltpu.make_async_copy(tbl.at[pl.ds(0, _CH)], buf_s, sem_s).wait()
        pltpu.make_async_copy(tbl.at[pl.ds(0, _CH)], buf_d, sem_d).wait()

    def compute_store(t, buf_s, buf_d, buf_u):
        def srow(r2, carry):
            for dr in range(2):
                r = 2 * r2 + dr
                for k in range(3):
                    buf_u[r, pl.ds(k * 16, 16)] = (
                        buf_s[r, pl.ds(k * 16, 16)]
                        + buf_d[r, pl.ds(48 + k * 16, 16)])
            return carry

        lax.fori_loop(0, _CH // 2, srow, 0)
        pltpu.sync_copy(buf_u, s_out.at[pl.ds((c0 + t) * _CH, _CH)])

    start(slot(0), buf_s0, buf_d0, sem_s0, sem_d0)

    def body(g, carry):
        j0 = 2 * g
        t0 = slot(j0)
        t1 = slot(j0 + 1)

        @pl.when(_slot_valid(c0, t1))
        def _():
            start(t1, buf_s1, buf_d1, sem_s1, sem_d1)

        @pl.when(_slot_valid(c0, t0))
        def _():
            wait(buf_s0, buf_d0, sem_s0, sem_d0)
            compute_store(t0, buf_s0, buf_d0, buf_u0)

        @pl.when((j0 + 2 < _NSL // 2) & _slot_valid(c0, slot(j0 + 2)))
        def _():
            start(slot(j0 + 2), buf_s0, buf_d0, sem_s0, sem_d0)

        @pl.when(_slot_valid(c0, t1))
        def _():
            wait(buf_s1, buf_d1, sem_s1, sem_d1)
            compute_store(t1, buf_s1, buf_d1, buf_u1)

        return carry

    lax.fori_loop(0, _NSL // 4, body, 0)

  return _sc_gather


_sc_gather_a = _make_sc_gather(0)
_sc_gather_b = _make_sc_gather(1)


@functools.partial(
    pl.kernel,
    mesh=_mesh,
    out_type=jax.ShapeDtypeStruct((2, _N, _D), jnp.float32),
    scratch_types=[
        pltpu.VMEM((_NSL, _CH), jnp.int32),
        pltpu.VMEM((_NSL, _CH), jnp.int32),
        pltpu.VMEM((_CH, _D), jnp.float32),
        pltpu.VMEM((_CH, _D), jnp.float32),
        pltpu.VMEM_SHARED((_N, _D), jnp.float32),
        pltpu.SemaphoreType.DMA,
        pltpu.SemaphoreType.DMA,
    ],
)
def _sc_scatter(gate2, a_tbl, src2, dst2, out, idx_s, idx_d,
                buf_g0, buf_a0, acc, sem_g0, sem_a0):
    cid = lax.axis_index("c")
    sid = lax.axis_index("s")
    wid = sid * 2 + cid
    c0 = wid * _NSL
    _preload_idx(src2, dst2, idx_s, idx_d, c0)

    def zrow(i, carry):
        for k in range(_D // 16):
            buf_g0[i, pl.ds(k * 16, 16)] = jnp.zeros((16,), jnp.float32)
        return carry

    lax.fori_loop(0, _CH, zrow, 0)
    r0 = sid * _RPS
    for t in range(4):
        pltpu.sync_copy(buf_g0, acc.at[pl.ds(r0 + t * _CH, _CH)])
    pltpu.sync_copy(buf_g0.at[pl.ds(0, _RPS - 4 * _CH)],
                    acc.at[pl.ds(r0 + 4 * _CH, _RPS - 4 * _CH)])

    @pl.when(sid == 0)
    def _():
        pltpu.sync_copy(buf_g0.at[pl.ds(0, _RTAIL)],
                        acc.at[pl.ds(16 * _RPS, _RTAIL)])

    plsc.subcore_barrier()

    pltpu.async_copy(a_tbl.at[idx_s.at[0]], buf_a0, sem_a0)
    pltpu.async_copy(gate2.at[pl.ds(c0 * _CH, _CH)], buf_g0, sem_g0)

    def body(t, carry):
        @pl.when(_slot_valid(c0, t))
        def _():
            pltpu.make_async_copy(a_tbl.at[pl.ds(0, _CH)], buf_a0,
                                  sem_a0).wait()
            pltpu.make_async_copy(gate2.at[pl.ds(0, _CH)], buf_g0,
                                  sem_g0).wait()

            def mrow(r2, c):
                for dr in range(2):
                    r = 2 * r2 + dr
                    for k in range(_D // 16):
                        sl = pl.ds(k * 16, 16)
                        buf_g0[r, sl] = buf_g0[r, sl] * buf_a0[r, sl]
                return c

            lax.fori_loop(0, _CH // 2, mrow, 0)

            @pl.when(_slot_valid(c0, t + 1))
            def _():
                pltpu.async_copy(a_tbl.at[idx_s.at[t + 1]], buf_a0, sem_a0)

            pltpu.sync_copy(buf_g0, acc.at[idx_d.at[t]], add=True)

            @pl.when(_slot_valid(c0, t + 1))
            def _():
                pltpu.async_copy(gate2.at[pl.ds((c0 + t + 1) * _CH, _CH)],
                                 buf_g0, sem_g0)

        return carry

    lax.fori_loop(0, _NSL, body, 0)

    plsc.subcore_barrier()
    pltpu.sync_copy(acc.at[pl.ds(r0, _RPS)], out.at[cid, pl.ds(r0, _RPS)])

    @pl.when(sid == 0)
    def _():
        pltpu.sync_copy(acc.at[pl.ds(16 * _RPS, _RTAIL)],
                        out.at[cid, pl.ds(16 * _RPS, _RTAIL)])


_BE = 2000
_BN = 1000


def _proj0_body(h_ref, w1_ref, gmid_ref, gbot_ref, a_ref, tbl_ref):
    h = h_ref[...]
    hs = h[:, :_NS]
    a_ref[...] = jnp.dot(h, w1_ref[...], preferred_element_type=jnp.float32)
    tbl_ref[:, :] = jnp.zeros(tbl_ref.shape, jnp.float32)
    tbl_ref[:, :48] = jnp.dot(hs, gmid_ref[...],
                              preferred_element_type=jnp.float32)
    tbl_ref[:, 48:96] = jnp.dot(hs, gbot_ref[...],
                                preferred_element_type=jnp.float32)


def _tc_proj0(h, w1, gmid, gbot):
    grid = (_N // _BN,)
    return pl.pallas_call(
        _proj0_body,
        grid=grid,
        in_specs=[
            pl.BlockSpec((_BN, _D), lambda i: (i, 0)),
            pl.BlockSpec((_D, _D), lambda i: (0, 0)),
            pl.BlockSpec((_NS, 48), lambda i: (0, 0)),
            pl.BlockSpec((_NS, 48), lambda i: (0, 0)),
        ],
        out_specs=[
            pl.BlockSpec((_BN, _D), lambda i: (i, 0)),
            pl.BlockSpec((_BN, _D), lambda i: (i, 0)),
        ],
        out_shape=[
            jax.ShapeDtypeStruct((_N, _D), jnp.float32),
            jax.ShapeDtypeStruct((_N, _D), jnp.float32),
        ],
    )(h, w1, gmid, gbot)


def _proj1_body(h_ref, p_ref, w1_ref, gmid_ref, gbot_ref,
                hout_ref, a_ref, tbl_ref):
    h = h_ref[...] + p_ref[0] + p_ref[1]
    hout_ref[...] = h
    hs = h[:, :_NS]
    a_ref[...] = jnp.dot(h, w1_ref[...], preferred_element_type=jnp.float32)
    tbl_ref[:, :] = jnp.zeros(tbl_ref.shape, jnp.float32)
    tbl_ref[:, :48] = jnp.dot(hs, gmid_ref[...],
                              preferred_element_type=jnp.float32)
    tbl_ref[:, 48:96] = jnp.dot(hs, gbot_ref[...],
                                preferred_element_type=jnp.float32)


def _tc_proj1(h, parts, w1, gmid, gbot):
    grid = (_N // _BN,)
    return pl.pallas_call(
        _proj1_body,
        grid=grid,
        in_specs=[
            pl.BlockSpec((_BN, _D), lambda i: (i, 0)),
            pl.BlockSpec((2, _BN, _D), lambda i: (0, i, 0)),
            pl.BlockSpec((_D, _D), lambda i: (0, 0)),
            pl.BlockSpec((_NS, 48), lambda i: (0, 0)),
            pl.BlockSpec((_NS, 48), lambda i: (0, 0)),
        ],
        out_specs=[
            pl.BlockSpec((_BN, _D), lambda i: (i, 0)),
            pl.BlockSpec((_BN, _D), lambda i: (i, 0)),
            pl.BlockSpec((_BN, _D), lambda i: (i, 0)),
        ],
        out_shape=[
            jax.ShapeDtypeStruct((_N, _D), jnp.float32),
            jax.ShapeDtypeStruct((_N, _D), jnp.float32),
            jax.ShapeDtypeStruct((_N, _D), jnp.float32),
        ],
    )(h, parts, w1, gmid, gbot)


def _edge_body(ea_ref, esh_ref, s_ref,
               ew1_ref, eb1_ref, ew2_ref, eb2_ref,
               gtop_ref, gb1_ref, gw2_ref, gb2_ref, w2_ref,
               gate2_ref):
    e = jnp.maximum(
        jnp.dot(ea_ref[...], ew1_ref[...], preferred_element_type=jnp.float32)
        + eb1_ref[...], 0.0)
    e = (jnp.dot(e, ew2_ref[...], preferred_element_type=jnp.float32)
         + eb2_ref[...])
    pre = (jnp.dot(e, gtop_ref[...], preferred_element_type=jnp.float32)
           + gb1_ref[...])
    u = jnp.maximum(pre + s_ref[...], 0.0)
    gate = (jnp.dot(u, gw2_ref[...], preferred_element_type=jnp.float32)
            + gb2_ref[...])
    shw = jnp.dot(esh_ref[...], w2_ref[...],
                  preferred_element_type=jnp.float32)
    gate2_ref[...] = gate * shw


_BES = 640


def _edge_body_aliased(ea_ref, esh_ref, s_ref,
                       ew1_ref, eb1_ref, ew2_ref, eb2_ref,
                       gtop_ref, gb1_ref, gw2_ref, gb2_ref, w2_ref,
                       prev_ref, gate2_ref):
    del prev_ref
    _edge_body(ea_ref, esh_ref, s_ref, ew1_ref, eb1_ref, ew2_ref, eb2_ref,
               gtop_ref, gb1_ref, gw2_ref, gb2_ref, w2_ref, gate2_ref)


def _tc_edge_stripe(k, edge_attr, edge_sh, s, ew1, eb1, ew2, eb2,
                    gtop, gb1, gw2, gb2, w2l, prev=None):
    grid = (_E // _BES // 2,)
    espec = lambda bs: pl.BlockSpec(bs, lambda i: (2 * i + k, 0))
    in_specs = [
        espec((_BES, _DE)),
        espec((_BES, _SH)),
        espec((_BES, 48)),
        pl.BlockSpec((_DE, _NS), lambda i: (0, 0)),
        pl.BlockSpec((1, _NS), lambda i: (0, 0)),
        pl.BlockSpec((_NS, _NS), lambda i: (0, 0)),
        pl.BlockSpec((1, _NS), lambda i: (0, 0)),
        pl.BlockSpec((_NS, 48), lambda i: (0, 0)),
        pl.BlockSpec((1, 48), lambda i: (0, 0)),
        pl.BlockSpec((48, _D), lambda i: (0, 0)),
        pl.BlockSpec((1, _D), lambda i: (0, 0)),
        pl.BlockSpec((_SH, _D), lambda i: (0, 0)),
    ]
    args = [edge_attr, edge_sh, s, ew1, eb1, ew2, eb2, gtop, gb1, gw2, gb2,
            w2l]
    if k == 0:
        body = _edge_body
        kwargs = {}
    else:
        body = _edge_body_aliased
        in_specs.append(pl.BlockSpec(memory_space=pl.ANY))
        args.append(prev)
        kwargs = {"input_output_aliases": {12: 0}}
    return pl.pallas_call(
        body,
        grid=grid,
        in_specs=in_specs,
        out_specs=pl.BlockSpec((_BES, _D), lambda i: (2 * i + k, 0)),
        out_shape=jax.ShapeDtypeStruct((_E, _D), jnp.float32),
        **kwargs,
    )(*args)


def _final_body(h_ref, p_ref, o_ref):
    o_ref[...] = h_ref[...] + p_ref[0] + p_ref[1]


def _tc_final(h, parts):
    grid = (_N // _BN,)
    return pl.pallas_call(
        _final_body,
        grid=grid,
        in_specs=[
            pl.BlockSpec((_BN, _D), lambda i: (i, 0)),
            pl.BlockSpec((2, _BN, _D), lambda i: (0, i, 0)),
        ],
        out_specs=pl.BlockSpec((_BN, _D), lambda i: (i, 0)),
        out_shape=jax.ShapeDtypeStruct((_N, _D), jnp.float32),
    )(h, parts)


def kernel(x, edge_attr, edge_sh, emb_w1, emb_b1, emb_w2, emb_b2,
           gate_w1, gate_b1, gate_w2, gate_b2, W1, W2, edge_index):
    ei = edge_index.astype(jnp.int32)
    pad = _NW * _NSL - _NCH
    src2 = jnp.pad(ei[0].reshape(_NCH, _CH), ((0, pad), (0, 0)))
    dst2 = jnp.pad(ei[1].reshape(_NCH, _CH), ((0, pad), (0, 0)))

    eb1 = emb_b1.reshape(1, _NS)
    eb2 = emb_b2.reshape(1, _NS)

    h = x
    parts = None
    for l in range(2):
        gtop = gate_w1[l, :_NS, :]
        gmid = gate_w1[l, _NS:2 * _NS, :]
        gbot = gate_w1[l, 2 * _NS:3 * _NS, :]
        if l == 0:
            a, tbl = _tc_proj0(h, W1[0], gmid, gbot)
        else:
            h, a, tbl = _tc_proj1(h, parts, W1[1], gmid, gbot)
        wl = (emb_w1, eb1, emb_w2, eb2, gtop, gate_b1[l].reshape(1, 48),
              gate_w2[l], gate_b2[l].reshape(1, _D), W2[l])
        s_a = _sc_gather_a(tbl, src2, dst2)
        s_b = _sc_gather_b(tbl, src2, dst2)
        g2 = _tc_edge_stripe(0, edge_attr, edge_sh, s_a, *wl)
        g2 = _tc_edge_stripe(1, edge_attr, edge_sh, s_b, *wl, prev=g2)
        parts = _sc_scatter(g2, a, src2, dst2)
    return _tc_final(h, parts)


# speedup vs baseline: 1.3058x; 1.3058x over previous
import functools

import jax
import jax.numpy as jnp
from jax import lax
from jax.experimental import pallas as pl
from jax.experimental.pallas import tpu as pltpu
from jax.experimental.pallas import tpu_sc as plsc

_NS = 16
_N = 10000
_E = 160000
_D = 128
_SH = 9
_DE = 64

_CH = 128
_NCH = _E // _CH
_NW = 32
_NSL = 40
_RPS = 624
_RTAIL = _N - 16 * _RPS

_mesh = plsc.VectorSubcoreMesh(core_axis_name="c", subcore_axis_name="s")


def _slot_valid(c0, t):
    return ((c0 + t) < _NCH) & (t < _NSL)


def _preload_idx(src2, dst2, idx_s, idx_d, c0):
    pltpu.sync_copy(src2.at[pl.ds(c0, _NSL)], idx_s)
    pltpu.sync_copy(dst2.at[pl.ds(c0, _NSL)], idx_d)


@functools.partial(
    pl.kernel,
    mesh=_mesh,
    out_type=jax.ShapeDtypeStruct((_E, 48), jnp.float32),
    scratch_types=[
        pltpu.VMEM((_NSL, _CH), jnp.int32),
        pltpu.VMEM((_NSL, _CH), jnp.int32),
        pltpu.VMEM((_CH, _D), jnp.float32),
        pltpu.VMEM((_CH, _D), jnp.float32),
        pltpu.VMEM((_CH, _D), jnp.float32),
        pltpu.VMEM((_CH, _D), jnp.float32),
        pltpu.VMEM((_CH, 48), jnp.float32),
        pltpu.VMEM((_CH, 48), jnp.float32),
        pltpu.SemaphoreType.DMA,
        pltpu.SemaphoreType.DMA,
        pltpu.SemaphoreType.DMA,
        pltpu.SemaphoreType.DMA,
    ],
)
def _sc_gather(tbl, src2, dst2, s_out, idx_s, idx_d,
               buf_s0, buf_s1, buf_d0, buf_d1, buf_u0, buf_u1,
               sem_s0, sem_s1, sem_d0, sem_d1):
    wid = lax.axis_index("s") * 2 + lax.axis_index("c")
    c0 = wid * _NSL
    _preload_idx(src2, dst2, idx_s, idx_d, c0)

    def start(t, buf_s, buf_d, sem_s, sem_d):
        pltpu.async_copy(tbl.at[idx_s.at[t]], buf_s, sem_s)
        pltpu.async_copy(tbl.at[idx_d.at[t]], buf_d, sem_d)

    def wait(buf_s, buf_d, sem_s, sem_d):
        pltpu.make_async_copy(tbl.at[pl.ds(0, _CH)], buf_s, sem_s).wait()
        pltpu.make_async_copy(tbl.at[pl.ds(0, _CH)], buf_d, sem_d).wait()

    def compute_store(t, buf_s, buf_d, buf_u):
        def srow(r2, carry):
            for dr in range(2):
                r = 2 * r2 + dr
                for k in range(3):
                    buf_u[r, pl.ds(k * 16, 16)] = (
                        buf_s[r, pl.ds(k * 16, 16)]
                        + buf_d[r, pl.ds(48 + k * 16, 16)])
            return carry

        lax.fori_loop(0, _CH // 2, srow, 0)
        pltpu.sync_copy(buf_u, s_out.at[pl.ds((c0 + t) * _CH, _CH)])

    start(0, buf_s0, buf_d0, sem_s0, sem_d0)

    def body(g, carry):
        t0 = 2 * g
        t1 = t0 + 1

        @pl.when(_slot_valid(c0, t1))
        def _():
            start(t1, buf_s1, buf_d1, sem_s1, sem_d1)

        @pl.when(_slot_valid(c0, t0))
        def _():
            wait(buf_s0, buf_d0, sem_s0, sem_d0)
            compute_store(t0, buf_s0, buf_d0, buf_u0)

        @pl.when(_slot_valid(c0, t0 + 2))
        def _():
            start(t0 + 2, buf_s0, buf_d0, sem_s0, sem_d0)

        @pl.when(_slot_valid(c0, t1))
        def _():
            wait(buf_s1, buf_d1, sem_s1, sem_d1)
            compute_store(t1, buf_s1, buf_d1, buf_u1)

        return carry

    lax.fori_loop(0, _NSL // 2, body, 0)


@functools.partial(
    pl.kernel,
    mesh=_mesh,
    out_type=jax.ShapeDtypeStruct((2, _N, _D), jnp.float32),
    scratch_types=[
        pltpu.VMEM((_NSL, _CH), jnp.int32),
        pltpu.VMEM((_NSL, _CH), jnp.int32),
        pltpu.VMEM((_CH, _D), jnp.float32),
        pltpu.VMEM((_CH, _D), jnp.float32),
        pltpu.VMEM_SHARED((_N, _D), jnp.float32),
        pltpu.SemaphoreType.DMA,
        pltpu.SemaphoreType.DMA,
    ],
)
def _sc_scatter(gate2, a_tbl, src2, dst2, out, idx_s, idx_d,
                buf_g0, buf_a0, acc, sem_g0, sem_a0):
    cid = lax.axis_index("c")
    sid = lax.axis_index("s")
    wid = sid * 2 + cid
    c0 = wid * _NSL
    _preload_idx(src2, dst2, idx_s, idx_d, c0)

    def zrow(i, carry):
        for k in range(_D // 16):
            buf_g0[i, pl.ds(k * 16, 16)] = jnp.zeros((16,), jnp.float32)
        return carry

    lax.fori_loop(0, _CH, zrow, 0)
    r0 = sid * _RPS
    for t in range(4):
        pltpu.sync_copy(buf_g0, acc.at[pl.ds(r0 + t * _CH, _CH)])
    pltpu.sync_copy(buf_g0.at[pl.ds(0, _RPS - 4 * _CH)],
                    acc.at[pl.ds(r0 + 4 * _CH, _RPS - 4 * _CH)])

    @pl.when(sid == 0)
    def _():
        pltpu.sync_copy(buf_g0.at[pl.ds(0, _RTAIL)],
                        acc.at[pl.ds(16 * _RPS, _RTAIL)])

    plsc.subcore_barrier()

    pltpu.async_copy(a_tbl.at[idx_s.at[0]], buf_a0, sem_a0)
    pltpu.async_copy(gate2.at[pl.ds(c0 * _CH, _CH)], buf_g0, sem_g0)

    def body(t, carry):
        @pl.when(_slot_valid(c0, t))
        def _():
            pltpu.make_async_copy(a_tbl.at[pl.ds(0, _CH)], buf_a0,
                                  sem_a0).wait()
            pltpu.make_async_copy(gate2.at[pl.ds(0, _CH)], buf_g0,
                                  sem_g0).wait()

            def mrow(r2, c):
                for dr in range(2):
                    r = 2 * r2 + dr
                    for k in range(_D // 16):
                        sl = pl.ds(k * 16, 16)
                        buf_g0[r, sl] = buf_g0[r, sl] * buf_a0[r, sl]
                return c

            lax.fori_loop(0, _CH // 2, mrow, 0)

            @pl.when(_slot_valid(c0, t + 1))
            def _():
                pltpu.async_copy(a_tbl.at[idx_s.at[t + 1]], buf_a0, sem_a0)

            pltpu.sync_copy(buf_g0, acc.at[idx_d.at[t]], add=True)

            @pl.when(_slot_valid(c0, t + 1))
            def _():
                pltpu.async_copy(gate2.at[pl.ds((c0 + t + 1) * _CH, _CH)],
                                 buf_g0, sem_g0)

        return carry

    lax.fori_loop(0, _NSL, body, 0)

    plsc.subcore_barrier()
    pltpu.sync_copy(acc.at[pl.ds(r0, _RPS)], out.at[cid, pl.ds(r0, _RPS)])

    @pl.when(sid == 0)
    def _():
        pltpu.sync_copy(acc.at[pl.ds(16 * _RPS, _RTAIL)],
                        out.at[cid, pl.ds(16 * _RPS, _RTAIL)])


_BE = 4000
_BN = 1000


def _proj0_body(h_ref, w1_ref, gmid_ref, gbot_ref, a_ref, tbl_ref):
    h = h_ref[...]
    hs = h[:, :_NS]
    a_ref[...] = jnp.dot(h, w1_ref[...], preferred_element_type=jnp.float32)
    tbl_ref[:, :] = jnp.zeros(tbl_ref.shape, jnp.float32)
    tbl_ref[:, :48] = jnp.dot(hs, gmid_ref[...],
                              preferred_element_type=jnp.float32)
    tbl_ref[:, 48:96] = jnp.dot(hs, gbot_ref[...],
                                preferred_element_type=jnp.float32)


def _tc_proj0(h, w1, gmid, gbot):
    grid = (_N // _BN,)
    return pl.pallas_call(
        _proj0_body,
        grid=grid,
        in_specs=[
            pl.BlockSpec((_BN, _D), lambda i: (i, 0)),
            pl.BlockSpec((_D, _D), lambda i: (0, 0)),
            pl.BlockSpec((_NS, 48), lambda i: (0, 0)),
            pl.BlockSpec((_NS, 48), lambda i: (0, 0)),
        ],
        out_specs=[
            pl.BlockSpec((_BN, _D), lambda i: (i, 0)),
            pl.BlockSpec((_BN, _D), lambda i: (i, 0)),
        ],
        out_shape=[
            jax.ShapeDtypeStruct((_N, _D), jnp.float32),
            jax.ShapeDtypeStruct((_N, _D), jnp.float32),
        ],
    )(h, w1, gmid, gbot)


def _proj1_body(h_ref, p_ref, w1_ref, gmid_ref, gbot_ref,
                hout_ref, a_ref, tbl_ref):
    h = h_ref[...] + p_ref[0] + p_ref[1]
    hout_ref[...] = h
    hs = h[:, :_NS]
    a_ref[...] = jnp.dot(h, w1_ref[...], preferred_element_type=jnp.float32)
    tbl_ref[:, :] = jnp.zeros(tbl_ref.shape, jnp.float32)
    tbl_ref[:, :48] = jnp.dot(hs, gmid_ref[...],
                              preferred_element_type=jnp.float32)
    tbl_ref[:, 48:96] = jnp.dot(hs, gbot_ref[...],
                                preferred_element_type=jnp.float32)


def _tc_proj1(h, parts, w1, gmid, gbot):
    grid = (_N // _BN,)
    return pl.pallas_call(
        _proj1_body,
        grid=grid,
        in_specs=[
            pl.BlockSpec((_BN, _D), lambda i: (i, 0)),
            pl.BlockSpec((2, _BN, _D), lambda i: (0, i, 0)),
            pl.BlockSpec((_D, _D), lambda i: (0, 0)),
            pl.BlockSpec((_NS, 48), lambda i: (0, 0)),
            pl.BlockSpec((_NS, 48), lambda i: (0, 0)),
        ],
        out_specs=[
            pl.BlockSpec((_BN, _D), lambda i: (i, 0)),
            pl.BlockSpec((_BN, _D), lambda i: (i, 0)),
            pl.BlockSpec((_BN, _D), lambda i: (i, 0)),
        ],
        out_shape=[
            jax.ShapeDtypeStruct((_N, _D), jnp.float32),
            jax.ShapeDtypeStruct((_N, _D), jnp.float32),
            jax.ShapeDtypeStruct((_N, _D), jnp.float32),
        ],
    )(h, parts, w1, gmid, gbot)


def _edge_body(ea_ref, esh_ref, s_ref,
               ew1_ref, eb1_ref, ew2_ref, eb2_ref,
               gtop_ref, gb1_ref, gw2_ref, gb2_ref, w2_ref,
               gate2_ref):
    e = jnp.maximum(
        jnp.dot(ea_ref[...], ew1_ref[...], preferred_element_type=jnp.float32)
        + eb1_ref[...], 0.0)
    e = (jnp.dot(e, ew2_ref[...], preferred_element_type=jnp.float32)
         + eb2_ref[...])
    pre = (jnp.dot(e, gtop_ref[...], preferred_element_type=jnp.float32)
           + gb1_ref[...])
    u = jnp.maximum(pre + s_ref[...], 0.0)
    gate = (jnp.dot(u, gw2_ref[...], preferred_element_type=jnp.float32)
            + gb2_ref[...])
    shw = jnp.dot(esh_ref[...], w2_ref[...],
                  preferred_element_type=jnp.float32)
    gate2_ref[...] = gate * shw


def _tc_edge(edge_attr, edge_sh, s, ew1, eb1, ew2, eb2,
             gtop, gb1, gw2, gb2, w2l):
    grid = (_E // _BE,)
    return pl.pallas_call(
        _edge_body,
        grid=grid,
        in_specs=[
            pl.BlockSpec((_BE, _DE), lambda i: (i, 0)),
            pl.BlockSpec((_BE, _SH), lambda i: (i, 0)),
            pl.BlockSpec((_BE, 48), lambda i: (i, 0)),
            pl.BlockSpec((_DE, _NS), lambda i: (0, 0)),
            pl.BlockSpec((1, _NS), lambda i: (0, 0)),
            pl.BlockSpec((_NS, _NS), lambda i: (0, 0)),
            pl.BlockSpec((1, _NS), lambda i: (0, 0)),
            pl.BlockSpec((_NS, 48), lambda i: (0, 0)),
            pl.BlockSpec((1, 48), lambda i: (0, 0)),
            pl.BlockSpec((48, _D), lambda i: (0, 0)),
            pl.BlockSpec((1, _D), lambda i: (0, 0)),
            pl.BlockSpec((_SH, _D), lambda i: (0, 0)),
        ],
        out_specs=pl.BlockSpec((_BE, _D), lambda i: (i, 0)),
        out_shape=jax.ShapeDtypeStruct((_E, _D), jnp.float32),
    )(edge_attr, edge_sh, s, ew1, eb1, ew2, eb2, gtop, gb1, gw2, gb2, w2l)


def _final_body(h_ref, p_ref, o_ref):
    o_ref[...] = h_ref[...] + p_ref[0] + p_ref[1]


def _tc_final(h, parts):
    grid = (_N // _BN,)
    return pl.pallas_call(
        _final_body,
        grid=grid,
        in_specs=[
            pl.BlockSpec((_BN, _D), lambda i: (i, 0)),
            pl.BlockSpec((2, _BN, _D), lambda i: (0, i, 0)),
        ],
        out_specs=pl.BlockSpec((_BN, _D), lambda i: (i, 0)),
        out_shape=jax.ShapeDtypeStruct((_N, _D), jnp.float32),
    )(h, parts)


def kernel(x, edge_attr, edge_sh, emb_w1, emb_b1, emb_w2, emb_b2,
           gate_w1, gate_b1, gate_w2, gate_b2, W1, W2, edge_index):
    ei = edge_index.astype(jnp.int32)
    pad = _NW * _NSL - _NCH
    src2 = jnp.pad(ei[0].reshape(_NCH, _CH), ((0, pad), (0, 0)))
    dst2 = jnp.pad(ei[1].reshape(_NCH, _CH), ((0, pad), (0, 0)))

    eb1 = emb_b1.reshape(1, _NS)
    eb2 = emb_b2.reshape(1, _NS)

    h = x
    parts = None
    for l in range(2):
        gtop = gate_w1[l, :_NS, :]
        gmid = gate_w1[l, _NS:2 * _NS, :]
        gbot = gate_w1[l, 2 * _NS:3 * _NS, :]
        if l == 0:
            a, tbl = _tc_proj0(h, W1[0], gmid, gbot)
        else:
            h, a, tbl = _tc_proj1(h, parts, W1[1], gmid, gbot)
        s = _sc_gather(tbl, src2, dst2)
        gate2 = _tc_edge(edge_attr, edge_sh, s, emb_w1, eb1, emb_w2, eb2,
                         gtop, gate_b1[l].reshape(1, 48),
                         gate_w2[l], gate_b2[l].reshape(1, _D), W2[l])
        parts = _sc_scatter(gate2, a, src2, dst2)
    return _tc_final(h, parts)


# speedup vs baseline: 1.3181x; 1.0094x over previous
import functools

import jax
import jax.numpy as jnp
from jax import lax
from jax.experimental import pallas as pl
from jax.experimental.pallas import tpu as pltpu
from jax.experimental.pallas import tpu_sc as plsc

_NS = 16
_N = 10000
_E = 160000
_D = 128
_SH = 9
_DE = 64

_CH = 128
_NCH = _E // _CH
_NW = 32
_NSL = 40
_RPS = 624
_RTAIL = _N - 16 * _RPS

_mesh = plsc.VectorSubcoreMesh(core_axis_name="c", subcore_axis_name="s")


def _slot_valid(c0, t):
    return ((c0 + t) < _NCH) & (t < _NSL)


def _preload_idx(src2, dst2, idx_s, idx_d, c0):
    pltpu.sync_copy(src2.at[pl.ds(c0, _NSL)], idx_s)
    pltpu.sync_copy(dst2.at[pl.ds(c0, _NSL)], idx_d)


@functools.partial(
    pl.kernel,
    mesh=_mesh,
    out_type=jax.ShapeDtypeStruct((_E, 48), jnp.float32),
    scratch_types=[
        pltpu.VMEM((_NSL, _CH), jnp.int32),
        pltpu.VMEM((_NSL, _CH), jnp.int32),
        pltpu.VMEM((_CH, _D), jnp.float32),
        pltpu.VMEM((_CH, _D), jnp.float32),
        pltpu.VMEM((_CH, _D), jnp.float32),
        pltpu.VMEM((_CH, _D), jnp.float32),
        pltpu.VMEM((_CH, 48), jnp.float32),
        pltpu.VMEM((_CH, 48), jnp.float32),
        pltpu.SemaphoreType.DMA,
        pltpu.SemaphoreType.DMA,
        pltpu.SemaphoreType.DMA,
        pltpu.SemaphoreType.DMA,
    ],
)
def _sc_gather(tbl, src2, dst2, s_out, idx_s, idx_d,
               buf_s0, buf_s1, buf_d0, buf_d1, buf_u0, buf_u1,
               sem_s0, sem_s1, sem_d0, sem_d1):
    wid = lax.axis_index("s") * 2 + lax.axis_index("c")
    c0 = wid * _NSL
    _preload_idx(src2, dst2, idx_s, idx_d, c0)

    def start(t, buf_s, buf_d, sem_s, sem_d):
        pltpu.async_copy(tbl.at[idx_s.at[t]], buf_s, sem_s)
        pltpu.async_copy(tbl.at[idx_d.at[t]], buf_d, sem_d)

    def wait(buf_s, buf_d, sem_s, sem_d):
        pltpu.make_async_copy(tbl.at[pl.ds(0, _CH)], buf_s, sem_s).wait()
        pltpu.make_async_copy(tbl.at[pl.ds(0, _CH)], buf_d, sem_d).wait()

    def compute_store(t, buf_s, buf_d, buf_u):
        def srow(r2, carry):
            for dr in range(2):
                r = 2 * r2 + dr
                for k in range(3):
                    buf_u[r, pl.ds(k * 16, 16)] = (
                        buf_s[r, pl.ds(k * 16, 16)]
                        + buf_d[r, pl.ds(48 + k * 16, 16)])
            return carry

        lax.fori_loop(0, _CH // 2, srow, 0)
        pltpu.sync_copy(buf_u, s_out.at[pl.ds((c0 + t) * _CH, _CH)])

    start(0, buf_s0, buf_d0, sem_s0, sem_d0)

    def body(g, carry):
        t0 = 2 * g
        t1 = t0 + 1

        @pl.when(_slot_valid(c0, t1))
        def _():
            start(t1, buf_s1, buf_d1, sem_s1, sem_d1)

        @pl.when(_slot_valid(c0, t0))
        def _():
            wait(buf_s0, buf_d0, sem_s0, sem_d0)
            compute_store(t0, buf_s0, buf_d0, buf_u0)

        @pl.when(_slot_valid(c0, t0 + 2))
        def _():
            start(t0 + 2, buf_s0, buf_d0, sem_s0, sem_d0)

        @pl.when(_slot_valid(c0, t1))
        def _():
            wait(buf_s1, buf_d1, sem_s1, sem_d1)
            compute_store(t1, buf_s1, buf_d1, buf_u1)

        return carry

    lax.fori_loop(0, _NSL // 2, body, 0)


@functools.partial(
    pl.kernel,
    mesh=_mesh,
    out_type=jax.ShapeDtypeStruct((2, _N, _D), jnp.float32),
    scratch_types=[
        pltpu.VMEM((_NSL, _CH), jnp.int32),
        pltpu.VMEM((_NSL, _CH), jnp.int32),
        pltpu.VMEM((_CH, _D), jnp.float32),
        pltpu.VMEM((_CH, _D), jnp.float32),
        pltpu.VMEM_SHARED((_N, _D), jnp.float32),
        pltpu.SemaphoreType.DMA,
        pltpu.SemaphoreType.DMA,
    ],
)
def _sc_scatter(gate2, a_tbl, src2, dst2, out, idx_s, idx_d,
                buf_g0, buf_a0, acc, sem_g0, sem_a0):
    cid = lax.axis_index("c")
    sid = lax.axis_index("s")
    wid = sid * 2 + cid
    c0 = wid * _NSL
    _preload_idx(src2, dst2, idx_s, idx_d, c0)

    def zrow(i, carry):
        for k in range(_D // 16):
            buf_g0[i, pl.ds(k * 16, 16)] = jnp.zeros((16,), jnp.float32)
        return carry

    lax.fori_loop(0, _CH, zrow, 0)
    r0 = sid * _RPS
    for t in range(4):
        pltpu.sync_copy(buf_g0, acc.at[pl.ds(r0 + t * _CH, _CH)])
    pltpu.sync_copy(buf_g0.at[pl.ds(0, _RPS - 4 * _CH)],
                    acc.at[pl.ds(r0 + 4 * _CH, _RPS - 4 * _CH)])

    @pl.when(sid == 0)
    def _():
        pltpu.sync_copy(buf_g0.at[pl.ds(0, _RTAIL)],
                        acc.at[pl.ds(16 * _RPS, _RTAIL)])

    plsc.subcore_barrier()

    pltpu.async_copy(a_tbl.at[idx_s.at[0]], buf_a0, sem_a0)
    pltpu.async_copy(gate2.at[pl.ds(c0 * _CH, _CH)], buf_g0, sem_g0)

    def body(t, carry):
        @pl.when(_slot_valid(c0, t))
        def _():
            pltpu.make_async_copy(a_tbl.at[pl.ds(0, _CH)], buf_a0,
                                  sem_a0).wait()
            pltpu.make_async_copy(gate2.at[pl.ds(0, _CH)], buf_g0,
                                  sem_g0).wait()

            def mrow(r2, c):
                for dr in range(2):
                    r = 2 * r2 + dr
                    for k in range(_D // 16):
                        sl = pl.ds(k * 16, 16)
                        buf_g0[r, sl] = buf_g0[r, sl] * buf_a0[r, sl]
                return c

            lax.fori_loop(0, _CH // 2, mrow, 0)

            @pl.when(_slot_valid(c0, t + 1))
            def _():
                pltpu.async_copy(a_tbl.at[idx_s.at[t + 1]], buf_a0, sem_a0)

            pltpu.sync_copy(buf_g0, acc.at[idx_d.at[t]], add=True)

            @pl.when(_slot_valid(c0, t + 1))
            def _():
                pltpu.async_copy(gate2.at[pl.ds((c0 + t + 1) * _CH, _CH)],
                                 buf_g0, sem_g0)

        return carry

    lax.fori_loop(0, _NSL, body, 0)

    plsc.subcore_barrier()
    pltpu.sync_copy(acc.at[pl.ds(r0, _RPS)], out.at[cid, pl.ds(r0, _RPS)])

    @pl.when(sid == 0)
    def _():
        pltpu.sync_copy(acc.at[pl.ds(16 * _RPS, _RTAIL)],
                        out.at[cid, pl.ds(16 * _RPS, _RTAIL)])


_BE = 8000
_BN = 1000


def _proj0_body(h_ref, w1_ref, gmid_ref, gbot_ref, a_ref, tbl_ref):
    h = h_ref[...]
    hs = h[:, :_NS]
    a_ref[...] = jnp.dot(h, w1_ref[...], preferred_element_type=jnp.float32)
    tbl_ref[:, :] = jnp.zeros(tbl_ref.shape, jnp.float32)
    tbl_ref[:, :48] = jnp.dot(hs, gmid_ref[...],
                              preferred_element_type=jnp.float32)
    tbl_ref[:, 48:96] = jnp.dot(hs, gbot_ref[...],
                                preferred_element_type=jnp.float32)


def _tc_proj0(h, w1, gmid, gbot):
    grid = (_N // _BN,)
    return pl.pallas_call(
        _proj0_body,
        grid=grid,
        in_specs=[
            pl.BlockSpec((_BN, _D), lambda i: (i, 0)),
            pl.BlockSpec((_D, _D), lambda i: (0, 0)),
            pl.BlockSpec((_NS, 48), lambda i: (0, 0)),
            pl.BlockSpec((_NS, 48), lambda i: (0, 0)),
        ],
        out_specs=[
            pl.BlockSpec((_BN, _D), lambda i: (i, 0)),
            pl.BlockSpec((_BN, _D), lambda i: (i, 0)),
        ],
        out_shape=[
            jax.ShapeDtypeStruct((_N, _D), jnp.float32),
            jax.ShapeDtypeStruct((_N, _D), jnp.float32),
        ],
    )(h, w1, gmid, gbot)


def _proj1_body(h_ref, p_ref, w1_ref, gmid_ref, gbot_ref,
                hout_ref, a_ref, tbl_ref):
    h = h_ref[...] + p_ref[0] + p_ref[1]
    hout_ref[...] = h
    hs = h[:, :_NS]
    a_ref[...] = jnp.dot(h, w1_ref[...], preferred_element_type=jnp.float32)
    tbl_ref[:, :] = jnp.zeros(tbl_ref.shape, jnp.float32)
    tbl_ref[:, :48] = jnp.dot(hs, gmid_ref[...],
                              preferred_element_type=jnp.float32)
    tbl_ref[:, 48:96] = jnp.dot(hs, gbot_ref[...],
                                preferred_element_type=jnp.float32)


def _tc_proj1(h, parts, w1, gmid, gbot):
    grid = (_N // _BN,)
    return pl.pallas_call(
        _proj1_body,
        grid=grid,
        in_specs=[
            pl.BlockSpec((_BN, _D), lambda i: (i, 0)),
            pl.BlockSpec((2, _BN, _D), lambda i: (0, i, 0)),
            pl.BlockSpec((_D, _D), lambda i: (0, 0)),
            pl.BlockSpec((_NS, 48), lambda i: (0, 0)),
            pl.BlockSpec((_NS, 48), lambda i: (0, 0)),
        ],
        out_specs=[
            pl.BlockSpec((_BN, _D), lambda i: (i, 0)),
            pl.BlockSpec((_BN, _D), lambda i: (i, 0)),
            pl.BlockSpec((_BN, _D), lambda i: (i, 0)),
        ],
        out_shape=[
            jax.ShapeDtypeStruct((_N, _D), jnp.float32),
            jax.ShapeDtypeStruct((_N, _D), jnp.float32),
            jax.ShapeDtypeStruct((_N, _D), jnp.float32),
        ],
    )(h, parts, w1, gmid, gbot)


def _edge_body(ea_ref, esh_ref, s_ref,
               ew1_ref, eb1_ref, ew2_ref, eb2_ref,
               gtop_ref, gb1_ref, gw2_ref, gb2_ref, w2_ref,
               gate2_ref):
    e = jnp.maximum(
        jnp.dot(ea_ref[...], ew1_ref[...], preferred_element_type=jnp.float32)
        + eb1_ref[...], 0.0)
    e = (jnp.dot(e, ew2_ref[...], preferred_element_type=jnp.float32)
         + eb2_ref[...])
    pre = (jnp.dot(e, gtop_ref[...], preferred_element_type=jnp.float32)
           + gb1_ref[...])
    u = jnp.maximum(pre + s_ref[...], 0.0)
    gate = (jnp.dot(u, gw2_ref[...], preferred_element_type=jnp.float32)
            + gb2_ref[...])
    shw = jnp.dot(esh_ref[...], w2_ref[...],
                  preferred_element_type=jnp.float32)
    gate2_ref[...] = gate * shw


def _tc_edge(edge_attr, edge_sh, s, ew1, eb1, ew2, eb2,
             gtop, gb1, gw2, gb2, w2l):
    grid = (_E // _BE,)
    return pl.pallas_call(
        _edge_body,
        grid=grid,
        in_specs=[
            pl.BlockSpec((_BE, _DE), lambda i: (i, 0)),
            pl.BlockSpec((_BE, _SH), lambda i: (i, 0)),
            pl.BlockSpec((_BE, 48), lambda i: (i, 0)),
            pl.BlockSpec((_DE, _NS), lambda i: (0, 0)),
            pl.BlockSpec((1, _NS), lambda i: (0, 0)),
            pl.BlockSpec((_NS, _NS), lambda i: (0, 0)),
            pl.BlockSpec((1, _NS), lambda i: (0, 0)),
            pl.BlockSpec((_NS, 48), lambda i: (0, 0)),
            pl.BlockSpec((1, 48), lambda i: (0, 0)),
            pl.BlockSpec((48, _D), lambda i: (0, 0)),
            pl.BlockSpec((1, _D), lambda i: (0, 0)),
            pl.BlockSpec((_SH, _D), lambda i: (0, 0)),
        ],
        out_specs=pl.BlockSpec((_BE, _D), lambda i: (i, 0)),
        out_shape=jax.ShapeDtypeStruct((_E, _D), jnp.float32),
    )(edge_attr, edge_sh, s, ew1, eb1, ew2, eb2, gtop, gb1, gw2, gb2, w2l)


def _final_body(h_ref, p_ref, o_ref):
    o_ref[...] = h_ref[...] + p_ref[0] + p_ref[1]


def _tc_final(h, parts):
    grid = (_N // _BN,)
    return pl.pallas_call(
        _final_body,
        grid=grid,
        in_specs=[
            pl.BlockSpec((_BN, _D), lambda i: (i, 0)),
            pl.BlockSpec((2, _BN, _D), lambda i: (0, i, 0)),
        ],
        out_specs=pl.BlockSpec((_BN, _D), lambda i: (i, 0)),
        out_shape=jax.ShapeDtypeStruct((_N, _D), jnp.float32),
    )(h, parts)


def kernel(x, edge_attr, edge_sh, emb_w1, emb_b1, emb_w2, emb_b2,
           gate_w1, gate_b1, gate_w2, gate_b2, W1, W2, edge_index):
    ei = edge_index.astype(jnp.int32)
    pad = _NW * _NSL - _NCH
    src2 = jnp.pad(ei[0].reshape(_NCH, _CH), ((0, pad), (0, 0)))
    dst2 = jnp.pad(ei[1].reshape(_NCH, _CH), ((0, pad), (0, 0)))

    eb1 = emb_b1.reshape(1, _NS)
    eb2 = emb_b2.reshape(1, _NS)

    h = x
    parts = None
    for l in range(2):
        gtop = gate_w1[l, :_NS, :]
        gmid = gate_w1[l, _NS:2 * _NS, :]
        gbot = gate_w1[l, 2 * _NS:3 * _NS, :]
        if l == 0:
            a, tbl = _tc_proj0(h, W1[0], gmid, gbot)
        else:
            h, a, tbl = _tc_proj1(h, parts, W1[1], gmid, gbot)
        s = _sc_gather(tbl, src2, dst2)
        gate2 = _tc_edge(edge_attr, edge_sh, s, emb_w1, eb1, emb_w2, eb2,
                         gtop, gate_b1[l].reshape(1, 48),
                         gate_w2[l], gate_b2[l].reshape(1, _D), W2[l])
        parts = _sc_scatter(gate2, a, src2, dst2)
    return _tc_final(h, parts)
